# baseline (speedup 1.0000x reference)
import functools

import jax
import jax.numpy as jnp
from jax import lax
from jax.experimental import pallas as pl
from jax.experimental.pallas import tpu as pltpu
from jax.experimental.pallas import tpu_sc as plsc

NE = 160000
NN = 10000
DL = 128
NG = 32
EPS = 1e-5
F32 = jnp.float32
PREC = None


def _bf(v):
    return v.astype(jnp.bfloat16)

SC_CORES = 2
SC_TILES = 16
EDGES_PER_TILE = NE // (SC_CORES * SC_TILES)
CHUNK = 96
NBUF = 4
NFULL = EDGES_PER_TILE // CHUNK
NQUAD = NFULL // NBUF
TAIL = EDGES_PER_TILE - NFULL * CHUNK
NPT = 624
NPT_LAST = NN - NPT * (SC_TILES - 1)

RE = 3200
RN = 2000


def _ln_rows(h, g, bt):
    m = jnp.mean(h, axis=-1, keepdims=True)
    d = h - m
    v = jnp.mean(d * d, axis=-1, keepdims=True)
    return d / jnp.sqrt(v + EPS) * g + bt


def _mlp(v, w_ref, b_ref, g_ref, bt_ref):
    h = jnp.dot(_bf(v), _bf(w_ref[...]), preferred_element_type=F32, precision=PREC) + b_ref[...]
    return _ln_rows(jnp.maximum(h, 0.0), g_ref[...], bt_ref[...])


def _edge_step_body(first, *refs):
    if first:
        (et_ref, gu_ref, gv_ref, we_ref, be_ref, ge_ref, bte_ref,
         a1_ref, a2_ref, d2_ref, s1_ref, s2_ref, bce_ref, gce_ref, btce_ref,
         w0_ref, b0_ref, g0_ref, bt0_ref, w1_ref, b1_ref, g1_ref, bt1_ref,
         wo_ref, bo_ref, ec_ref, oe_ref, p2_ref) = refs
        h = lax.dot_general(_bf(et_ref[...]), _bf(we_ref[...]), (((0,), (0,)), ((), ())),
                            preferred_element_type=F32, precision=PREC) + be_ref[...]
        el0 = _ln_rows(jnp.maximum(h, 0.0), ge_ref[...], bte_ref[...])
        t1 = jnp.dot(_bf(el0), _bf(a1_ref[...]), preferred_element_type=F32, precision=PREC)
        pre = t1 + jnp.dot(_bf(el0), _bf(a2_ref[...]), preferred_element_type=F32, precision=PREC)
    else:
        (base_ref, gu_ref, gv_ref, d2_ref, s1_ref, s2_ref,
         bce_ref, gce_ref, btce_ref,
         w0_ref, b0_ref, g0_ref, bt0_ref, w1_ref, b1_ref, g1_ref, bt1_ref,
         wo_ref, bo_ref, ec_ref, oe_ref) = refs
        pre = base_ref[...]
    bias = s1_ref[...] * d2_ref[0:1, :] + s2_ref[...] * d2_ref[1:2, :]
    pre = pre + gu_ref[...] + gv_ref[...] + bias + bce_ref[...]
    ec = _ln_rows(jnp.maximum(pre, 0.0), gce_ref[...], btce_ref[...])
    ec_ref[...] = ec
    d0 = _mlp(ec, w0_ref, b0_ref, g0_ref, bt0_ref)
    d1 = _mlp(d0, w1_ref, b1_ref, g1_ref, bt1_ref)
    oe_ref[...] = lax.dot_general(_bf(wo_ref[...]), _bf(d1), (((1,), (1,)), ((), ())),
                                  preferred_element_type=F32,
                                  precision=PREC) + bo_ref[...]
    if first:
        p2_ref[...] = t1 + jnp.dot(_bf(ec), _bf(a2_ref[...]), preferred_element_type=F32, precision=PREC)


def _enc_node_body(x_ref, w_ref, b_ref, g_ref, bt_ref,
                   b1_ref, b2_ref, c1_ref, c2_ref, n1_ref, n2_ref,
                   us1_ref, vs1_ref, u0_ref, v0_ref, xp1_ref, x0_ref):
    xl0 = _mlp(x_ref[...], w_ref, b_ref, g_ref, bt_ref)

    def dot(w_ref):
        return jnp.dot(_bf(xl0), _bf(w_ref[...]), preferred_element_type=F32,
                       precision=PREC)

    u0 = dot(b1_ref)
    v0 = dot(c1_ref)
    x0 = dot(n1_ref)
    u0_ref[...] = u0
    v0_ref[...] = v0
    x0_ref[...] = x0
    us1_ref[...] = u0 + dot(b2_ref)
    vs1_ref[...] = v0 + dot(c2_ref)
    xp1_ref[...] = x0 + dot(n2_ref)


def _node_step_body(first, *refs):
    if first:
        (base_ref, a0_ref, a1_ref, n3_ref, nd_ref, s1_ref, s2_ref,
         bcn_ref, gcn_ref, btcn_ref, w0_ref, b0_ref, g0_ref, bt0_ref,
         w1_ref, b1_ref, g1_ref, bt1_ref, wo_ref, bo_ref,
         u0_ref, v0_ref, b2w_ref, c2w_ref,
         xout_ref, ox_ref, us_ref, vs_ref) = refs
        pre = base_ref[...]
    else:
        (base_ref, xc_ref, n2_ref, a0_ref, a1_ref, n3_ref, nd_ref,
         s1_ref, s2_ref, bcn_ref, gcn_ref, btcn_ref,
         w0_ref, b0_ref, g0_ref, bt0_ref, w1_ref, b1_ref, g1_ref, bt1_ref,
         wo_ref, bo_ref, ox_ref) = refs
        pre = base_ref[...] + jnp.dot(_bf(xc_ref[...]), _bf(n2_ref[...]),
                                      preferred_element_type=F32, precision=PREC)
    agg = a0_ref[...] + a1_ref[...]
    bias = s1_ref[...] * nd_ref[0:1, :] + s2_ref[...] * nd_ref[1:2, :]
    pre = pre + jnp.dot(_bf(agg), _bf(n3_ref[...]), preferred_element_type=F32, precision=PREC)
    pre = pre + bias + bcn_ref[...]
    xc = _ln_rows(jnp.maximum(pre, 0.0), gcn_ref[...], btcn_ref[...])
    d0 = _mlp(xc, w0_ref, b0_ref, g0_ref, bt0_ref)
    d1 = _mlp(d0, w1_ref, b1_ref, g1_ref, bt1_ref)
    ox_ref[...] = jnp.dot(_bf(d1), _bf(wo_ref[...]), preferred_element_type=F32, precision=PREC) + bo_ref[...]
    if first:
        xout_ref[...] = xc
        us_ref[...] = u0_ref[...] + jnp.dot(_bf(xc), _bf(b2w_ref[...]),
                                            preferred_element_type=F32, precision=PREC)
        vs_ref[...] = v0_ref[...] + jnp.dot(_bf(xc), _bf(c2w_ref[...]),
                                            preferred_element_type=F32, precision=PREC)


def _og_body(btd_ref, wg_ref, bg_ref, og_ref):
    val = btd_ref[...] * wg_ref[...] + bg_ref[...]
    og_ref[...] = jnp.broadcast_to(val, (NG, 1))


def _sc_gather(us, vs, src, dst):
    mesh = plsc.VectorSubcoreMesh(core_axis_name="c", subcore_axis_name="s")

    def body(us_hbm, vs_hbm, src_hbm, dst_hbm, gu_hbm, gv_hbm, *scr):
        idxu = scr[0:NBUF]
        idxv = scr[NBUF:2 * NBUF]
        rowsu = scr[2 * NBUF:3 * NBUF]
        rowsv = scr[3 * NBUF:4 * NBUF]
        gsem = scr[4 * NBUF:5 * NBUF]
        wsem = scr[5 * NBUF:6 * NBUF]
        cid = lax.axis_index("c")
        sid = lax.axis_index("s")
        base = cid * (SC_TILES * EDGES_PER_TILE) + sid * EDGES_PER_TILE

        def g_start(b, off):
            pltpu.sync_copy(src_hbm.at[pl.ds(off, CHUNK)], idxu[b])
            pltpu.sync_copy(dst_hbm.at[pl.ds(off, CHUNK)], idxv[b])
            pltpu.async_copy(us_hbm.at[idxu[b]], rowsu[b], gsem[b])
            pltpu.async_copy(vs_hbm.at[idxv[b]], rowsv[b], gsem[b])

        def g_wait(b):
            pltpu.make_async_copy(us_hbm.at[idxu[b]], rowsu[b], gsem[b]).wait()
            pltpu.make_async_copy(vs_hbm.at[idxv[b]], rowsv[b], gsem[b]).wait()

        def w_start(b, off):
            pltpu.async_copy(rowsu[b], gu_hbm.at[pl.ds(off, CHUNK)], wsem[b])
            pltpu.async_copy(rowsv[b], gv_hbm.at[pl.ds(off, CHUNK)], wsem[b])

        def w_wait(b):
            pltpu.make_async_copy(rowsu[b], gu_hbm.at[pl.ds(0, CHUNK)],
                                  wsem[b]).wait()
            pltpu.make_async_copy(rowsv[b], gv_hbm.at[pl.ds(0, CHUNK)],
                                  wsem[b]).wait()

        for b in range(NBUF):
            g_start(b, base + b * CHUNK)

        def quad(k, carry):
            for b in range(NBUF):
                g_wait(b)
                w_start(b, base + ((k - 1) * NBUF + b) * CHUNK)
            for b in range(NBUF):
                w_wait(b)
                g_start(b, base + (k * NBUF + b) * CHUNK)
            return carry

        lax.fori_loop(1, NQUAD, quad, 0)
        for b in range(NBUF):
            g_wait(b)
            w_start(b, base + ((NQUAD - 1) * NBUF + b) * CHUNK)
        offt = base + NFULL * CHUNK
        w_wait(0)
        pltpu.sync_copy(src_hbm.at[pl.ds(offt, TAIL)], idxu[0].at[pl.ds(0, TAIL)])
        pltpu.sync_copy(dst_hbm.at[pl.ds(offt, TAIL)], idxv[0].at[pl.ds(0, TAIL)])
        cu = pltpu.async_copy(us_hbm.at[idxu[0].at[pl.ds(0, TAIL)]],
                              rowsu[0].at[pl.ds(0, TAIL)], gsem[0])
        cv = pltpu.async_copy(vs_hbm.at[idxv[0].at[pl.ds(0, TAIL)]],
                              rowsv[0].at[pl.ds(0, TAIL)], gsem[0])
        cu.wait()
        cv.wait()
        pltpu.sync_copy(rowsu[0].at[pl.ds(0, TAIL)], gu_hbm.at[pl.ds(offt, TAIL)])
        pltpu.sync_copy(rowsv[0].at[pl.ds(0, TAIL)], gv_hbm.at[pl.ds(offt, TAIL)])
        for b in range(1, NBUF):
            w_wait(b)

    return pl.kernel(
        body,
        out_type=(jax.ShapeDtypeStruct((NE, DL), F32),
                  jax.ShapeDtypeStruct((NE, DL), F32)),
        mesh=mesh,
        scratch_types=(
            [pltpu.VMEM((CHUNK,), jnp.int32)] * (2 * NBUF)
            + [pltpu.VMEM((CHUNK, DL), F32)] * (2 * NBUF)
            + [pltpu.SemaphoreType.DMA] * (2 * NBUF)
        ),
    )(us, vs, src, dst)


def _sc_scatter_add(ec, dst, zeros):
    mesh = plsc.VectorSubcoreMesh(core_axis_name="c", subcore_axis_name="s")

    def body(ec_hbm, dst_hbm, z_hbm, out_hbm, shared, *scr):
        idxb = scr[0:NBUF]
        rows = scr[NBUF:2 * NBUF]
        rsem = scr[2 * NBUF:3 * NBUF]
        ssem = scr[3 * NBUF:4 * NBUF]
        idxt, rowst = scr[4 * NBUF], scr[4 * NBUF + 1]
        cid = lax.axis_index("c")
        sid = lax.axis_index("s")

        @pl.when(sid < SC_TILES - 1)
        def _():
            pltpu.sync_copy(z_hbm.at[pl.ds(sid * NPT, NPT)],
                            shared.at[pl.ds(sid * NPT, NPT)])

        @pl.when(sid == SC_TILES - 1)
        def _():
            pltpu.sync_copy(z_hbm.at[pl.ds((SC_TILES - 1) * NPT, NPT_LAST)],
                            shared.at[pl.ds((SC_TILES - 1) * NPT, NPT_LAST)])

        plsc.subcore_barrier()
        base = cid * (SC_TILES * EDGES_PER_TILE) + sid * EDGES_PER_TILE

        def r_start(b, off):
            pltpu.sync_copy(dst_hbm.at[pl.ds(off, CHUNK)], idxb[b])
            pltpu.async_copy(ec_hbm.at[pl.ds(off, CHUNK)], rows[b], rsem[b])

        def r_wait(b):
            pltpu.make_async_copy(ec_hbm.at[pl.ds(0, CHUNK)], rows[b],
                                  rsem[b]).wait()

        def s_start(b):
            pltpu.async_copy(rows[b], shared.at[idxb[b]], ssem[b], add=True)

        def s_wait(b):
            pltpu.make_async_copy(rows[b], shared.at[idxb[b]], ssem[b]).wait()

        for b in range(NBUF):
            r_start(b, base + b * CHUNK)

        def quad(k, carry):
            for b in range(NBUF):
                r_wait(b)
                s_start(b)
            for b in range(NBUF):
                s_wait(b)
                r_start(b, base + (k * NBUF + b) * CHUNK)
            return carry

        lax.fori_loop(1, NQUAD, quad, 0)
        for b in range(NBUF):
            r_wait(b)
            s_start(b)
        for b in range(NBUF):
            s_wait(b)
        off = base + NFULL * CHUNK
        pltpu.sync_copy(dst_hbm.at[pl.ds(off, TAIL)], idxt)
        pltpu.sync_copy(ec_hbm.at[pl.ds(off, TAIL)], rowst)
        pltpu.sync_copy(rowst, shared.at[idxt], add=True)
        plsc.subcore_barrier()

        @pl.when(sid < SC_TILES - 1)
        def _():
            pltpu.sync_copy(shared.at[pl.ds(sid * NPT, NPT)],
                            out_hbm.at[pl.ds(cid * NN + sid * NPT, NPT)])

        @pl.when(sid == SC_TILES - 1)
        def _():
            pltpu.sync_copy(
                shared.at[pl.ds((SC_TILES - 1) * NPT, NPT_LAST)],
                out_hbm.at[pl.ds(cid * NN + (SC_TILES - 1) * NPT, NPT_LAST)])

    return pl.kernel(
        body,
        out_type=jax.ShapeDtypeStruct((2 * NN, DL), F32),
        mesh=mesh,
        scratch_types=(
            [pltpu.VMEM_SHARED((NN, DL), F32)]
            + [pltpu.VMEM((CHUNK,), jnp.int32)] * NBUF
            + [pltpu.VMEM((CHUNK, DL), F32)] * NBUF
            + [pltpu.SemaphoreType.DMA] * (2 * NBUF)
            + [pltpu.VMEM((TAIL,), jnp.int32),
               pltpu.VMEM((TAIL, DL), F32)]
        ),
    )(ec, dst, zeros)


def _row(a):
    return a.reshape(1, -1)


def _wspec():
    return pl.BlockSpec((DL, DL), lambda i: (0, 0))


def _rspec():
    return pl.BlockSpec((1, DL), lambda i: (0, 0))


def _sspec():
    return pl.BlockSpec((1, 1), lambda i: (0, 0))


def kernel(x, e, g, params, edges, node_idx, edge_idx, steps):
    del g, node_idx, edge_idx, steps
    p = params
    src = edges[0]
    dst = edges[1]

    Wce = p['core_edge']['W']
    A1, A2 = Wce[0:128], Wce[128:256]
    B1, B2 = Wce[256:384], Wce[384:512]
    C1, C2 = Wce[512:640], Wce[640:768]
    D2 = Wce[768:770]
    Wcn = p['core_node']['W']
    N1, N2, N3 = Wcn[0:128], Wcn[128:256], Wcn[256:384]
    ND = Wcn[384:386]

    beta_e = p['enc_glob']['bt'].reshape(1, 1)
    beta_c = p['core_glob']['bt'].reshape(1, 1)

    GE = NE // RE
    GN = NN // RN
    en = p['enc_node']
    us, vs, u0, v0, xp1, x0 = pl.pallas_call(
        _enc_node_body,
        grid=(GN,),
        in_specs=[pl.BlockSpec((RN, DL), lambda i: (i, 0)),
                  _wspec(), _rspec(), _rspec(), _rspec(),
                  _wspec(), _wspec(), _wspec(), _wspec(), _wspec(), _wspec()],
        out_specs=[pl.BlockSpec((RN, DL), lambda i: (i, 0))] * 6,
        out_shape=[jax.ShapeDtypeStruct((NN, DL), F32)] * 6,
    )(x, en['W'], _row(en['b']), _row(en['g']), _row(en['bt']),
      B1, B2, C1, C2, N1, N2)

    ee = p['enc_edge']
    zeros = jnp.zeros((NN, DL), F32)
    ce = p['core_edge']
    de0, de1, oe_p = p['dec_edge0'], p['dec_edge1'], p['out_edge']
    cn = p['core_node']
    dn0, dn1, on_p = p['dec_node0'], p['dec_node1'], p['out_node']

    espec = pl.BlockSpec((RE, DL), lambda i: (i, 0))

    def edge_step(first, gu, gv, scal2, base=None):
        if first:
            in_specs = [pl.BlockSpec((16, RE), lambda i: (0, i)), espec, espec,
                        pl.BlockSpec((16, DL), lambda i: (0, 0)),
                        _rspec(), _rspec(), _rspec(), _wspec(), _wspec()]
            args = [e.T, gu, gv, ee['W'], _row(ee['b']), _row(ee['g']),
                    _row(ee['bt']), A1, A2]
        else:
            in_specs = [espec, espec, espec]
            args = [base, gu, gv]
        in_specs += [pl.BlockSpec((2, DL), lambda i: (0, 0)),
                     _sspec(), _sspec(), _rspec(), _rspec(), _rspec(),
                     _wspec(), _rspec(), _rspec(), _rspec(),
                     _wspec(), _rspec(), _rspec(), _rspec(),
                     _rspec(), _sspec()]
        args += [D2, beta_e, scal2,
                 _row(ce['b']), _row(ce['g']), _row(ce['bt']),
                 de0['W'], _row(de0['b']), _row(de0['g']), _row(de0['bt']),
                 de1['W'], _row(de1['b']), _row(de1['g']), _row(de1['bt']),
                 oe_p['W'].reshape(1, DL), oe_p['b'].reshape(1, 1)]
        out_specs = [espec, pl.BlockSpec((1, RE), lambda i: (0, i))]
        out_shape = [jax.ShapeDtypeStruct((NE, DL), F32),
                     jax.ShapeDtypeStruct((1, NE), F32)]
        if first:
            out_specs += [espec]
            out_shape += [jax.ShapeDtypeStruct((NE, DL), F32)]
        return pl.pallas_call(
            functools.partial(_edge_step_body, first),
            grid=(GE,),
            in_specs=in_specs,
            out_specs=out_specs,
            out_shape=out_shape,
        )(*args)

    def node_step(first, base, aggp, scal2, xcur=None):
        nspec = pl.BlockSpec((RN, DL), lambda i: (i, 0))
        in_specs = [nspec]
        args = [base]
        if not first:
            in_specs += [nspec, _wspec()]
            args += [xcur, N2]
        in_specs += [nspec, nspec, _wspec(),
                     pl.BlockSpec((2, DL), lambda i: (0, 0)),
                     _sspec(), _sspec(),
                     _rspec(), _rspec(), _rspec(),
                     _wspec(), _rspec(), _rspec(), _rspec(),
                     _wspec(), _rspec(), _rspec(), _rspec(),
                     pl.BlockSpec((DL, 1), lambda i: (0, 0)), _sspec()]
        args += [aggp[:NN], aggp[NN:], N3, ND, beta_e, scal2,
                 _row(cn['b']), _row(cn['g']), _row(cn['bt']),
                 dn0['W'], _row(dn0['b']), _row(dn0['g']), _row(dn0['bt']),
                 dn1['W'], _row(dn1['b']), _row(dn1['g']), _row(dn1['bt']),
                 on_p['W'], on_p['b'].reshape(1, 1)]
        if first:
            in_specs += [nspec, nspec, _wspec(), _wspec()]
            args += [u0, v0, B2, C2]
            out_specs = [nspec, pl.BlockSpec((RN, 1), lambda i: (i, 0)),
                         nspec, nspec]
            out_shape = [jax.ShapeDtypeStruct((NN, DL), F32),
                         jax.ShapeDtypeStruct((NN, 1), F32),
                         jax.ShapeDtypeStruct((NN, DL), F32),
                         jax.ShapeDtypeStruct((NN, DL), F32)]
        else:
            out_specs = [pl.BlockSpec((RN, 1), lambda i: (i, 0))]
            out_shape = [jax.ShapeDtypeStruct((NN, 1), F32)]
        return pl.pallas_call(
            functools.partial(_node_step_body, first),
            grid=(GN,),
            in_specs=in_specs,
            out_specs=out_specs,
            out_shape=out_shape,
        )(*args)

    gu1, gv1 = _sc_gather(us, vs, src, dst)
    ecur2, oe1t, p2 = edge_step(True, gu1, gv1, beta_e)
    aggp1 = _sc_scatter_add(ecur2, dst, zeros)
    xcur2, ox1, us2, vs2 = node_step(True, xp1, aggp1, beta_e)

    gu2, gv2 = _sc_gather(us2, vs2, src, dst)
    ecur3, oe2t = edge_step(False, gu2, gv2, beta_c, base=p2)
    aggp2 = _sc_scatter_add(ecur3, dst, zeros)
    (ox2,) = node_step(False, x0, aggp2, beta_c, xcur=xcur2)
    oe1 = oe1t.reshape(NE, 1)
    oe2 = oe2t.reshape(NE, 1)

    og_p = p['out_glob']
    og = pl.pallas_call(
        _og_body,
        in_specs=[pl.BlockSpec(memory_space=pltpu.VMEM)] * 3,
        out_shape=jax.ShapeDtypeStruct((NG, 1), F32),
    )(p['dec_glob']['bt'].reshape(1, 1), og_p['W'], og_p['b'].reshape(1, 1))

    return (ox1, oe1, og, ox2, oe2, og)


# speedup vs baseline: 1.0033x; 1.0033x over previous
import functools

import jax
import jax.numpy as jnp
from jax import lax
from jax.experimental import pallas as pl
from jax.experimental.pallas import tpu as pltpu
from jax.experimental.pallas import tpu_sc as plsc

NE = 160000
NN = 10000
DL = 128
NG = 32
EPS = 1e-5
F32 = jnp.float32
PREC = None

SC_CORES = 2
SC_TILES = 16
EDGES_PER_TILE = NE // (SC_CORES * SC_TILES)
CHUNK = 96
NBUF = 4
NFULL = EDGES_PER_TILE // CHUNK
NQUAD = NFULL // NBUF
TAIL = EDGES_PER_TILE - NFULL * CHUNK
NPT = 624
NPT_LAST = NN - NPT * (SC_TILES - 1)

RE = 3200
RN = 2000


def _ln_rows(h, g, bt):
    m = jnp.mean(h, axis=-1, keepdims=True)
    d = h - m
    v = jnp.mean(d * d, axis=-1, keepdims=True)
    return d / jnp.sqrt(v + EPS) * g + bt


def _mlp(v, w_ref, b_ref, g_ref, bt_ref):
    h = jnp.dot((v), (w_ref[...]), preferred_element_type=F32, precision=PREC) + b_ref[...]
    return _ln_rows(jnp.maximum(h, 0.0), g_ref[...], bt_ref[...])


def _edge_step_body(first, *refs):
    if first:
        (et_ref, gu_ref, gv_ref, we_ref, be_ref, ge_ref, bte_ref,
         a1_ref, a2_ref, d2_ref, s1_ref, s2_ref, bce_ref, gce_ref, btce_ref,
         w0_ref, b0_ref, g0_ref, bt0_ref, w1_ref, b1_ref, g1_ref, bt1_ref,
         wo_ref, bo_ref, ec_ref, oe_ref, p2_ref) = refs
        h = lax.dot_general((et_ref[...]), (we_ref[...]), (((0,), (0,)), ((), ())),
                            preferred_element_type=F32, precision=PREC) + be_ref[...]
        el0 = _ln_rows(jnp.maximum(h, 0.0), ge_ref[...], bte_ref[...])
        t1 = jnp.dot((el0), (a1_ref[...]), preferred_element_type=F32, precision=PREC)
        pre = t1 + jnp.dot((el0), (a2_ref[...]), preferred_element_type=F32, precision=PREC)
    else:
        (base_ref, gu_ref, gv_ref, d2_ref, s1_ref, s2_ref,
         bce_ref, gce_ref, btce_ref,
         w0_ref, b0_ref, g0_ref, bt0_ref, w1_ref, b1_ref, g1_ref, bt1_ref,
         wo_ref, bo_ref, ec_ref, oe_ref) = refs
        pre = base_ref[...]
    bias = s1_ref[...] * d2_ref[0:1, :] + s2_ref[...] * d2_ref[1:2, :]
    pre = pre + gu_ref[...] + gv_ref[...] + bias + bce_ref[...]
    ec = _ln_rows(jnp.maximum(pre, 0.0), gce_ref[...], btce_ref[...])
    ec_ref[...] = ec
    d0 = _mlp(ec, w0_ref, b0_ref, g0_ref, bt0_ref)
    d1 = _mlp(d0, w1_ref, b1_ref, g1_ref, bt1_ref)
    oe_ref[...] = lax.dot_general((wo_ref[...]), (d1), (((1,), (1,)), ((), ())),
                                  preferred_element_type=F32,
                                  precision=PREC) + bo_ref[...]
    if first:
        p2_ref[...] = t1 + jnp.dot((ec), (a2_ref[...]), preferred_element_type=F32, precision=PREC)


def _enc_node_body(x_ref, w_ref, b_ref, g_ref, bt_ref,
                   b1_ref, b2_ref, c1_ref, c2_ref, n1_ref, n2_ref,
                   us1_ref, vs1_ref, u0_ref, v0_ref, xp1_ref, x0_ref):
    xl0 = _mlp(x_ref[...], w_ref, b_ref, g_ref, bt_ref)

    def dot(w_ref):
        return jnp.dot((xl0), (w_ref[...]), preferred_element_type=F32,
                       precision=PREC)

    u0 = dot(b1_ref)
    v0 = dot(c1_ref)
    x0 = dot(n1_ref)
    u0_ref[...] = u0
    v0_ref[...] = v0
    x0_ref[...] = x0
    us1_ref[...] = u0 + dot(b2_ref)
    vs1_ref[...] = v0 + dot(c2_ref)
    xp1_ref[...] = x0 + dot(n2_ref)


def _node_step_body(first, *refs):
    if first:
        (base_ref, a0_ref, a1_ref, n3_ref, nd_ref, s1_ref, s2_ref,
         bcn_ref, gcn_ref, btcn_ref, w0_ref, b0_ref, g0_ref, bt0_ref,
         w1_ref, b1_ref, g1_ref, bt1_ref, wo_ref, bo_ref,
         u0_ref, v0_ref, b2w_ref, c2w_ref,
         xout_ref, ox_ref, us_ref, vs_ref) = refs
        pre = base_ref[...]
    else:
        (base_ref, xc_ref, n2_ref, a0_ref, a1_ref, n3_ref, nd_ref,
         s1_ref, s2_ref, bcn_ref, gcn_ref, btcn_ref,
         w0_ref, b0_ref, g0_ref, bt0_ref, w1_ref, b1_ref, g1_ref, bt1_ref,
         wo_ref, bo_ref, ox_ref) = refs
        pre = base_ref[...] + jnp.dot((xc_ref[...]), (n2_ref[...]),
                                      preferred_element_type=F32, precision=PREC)
    agg = a0_ref[...] + a1_ref[...]
    bias = s1_ref[...] * nd_ref[0:1, :] + s2_ref[...] * nd_ref[1:2, :]
    pre = pre + jnp.dot((agg), (n3_ref[...]), preferred_element_type=F32, precision=PREC)
    pre = pre + bias + bcn_ref[...]
    xc = _ln_rows(jnp.maximum(pre, 0.0), gcn_ref[...], btcn_ref[...])
    d0 = _mlp(xc, w0_ref, b0_ref, g0_ref, bt0_ref)
    d1 = _mlp(d0, w1_ref, b1_ref, g1_ref, bt1_ref)
    ox_ref[...] = jnp.dot((d1), (wo_ref[...]), preferred_element_type=F32, precision=PREC) + bo_ref[...]
    if first:
        xout_ref[...] = xc
        us_ref[...] = u0_ref[...] + jnp.dot((xc), (b2w_ref[...]),
                                            preferred_element_type=F32, precision=PREC)
        vs_ref[...] = v0_ref[...] + jnp.dot((xc), (c2w_ref[...]),
                                            preferred_element_type=F32, precision=PREC)


def _og_body(btd_ref, wg_ref, bg_ref, og_ref):
    val = btd_ref[...] * wg_ref[...] + bg_ref[...]
    og_ref[...] = jnp.broadcast_to(val, (NG, 1))


def _sc_gather(us, vs, src, dst):
    mesh = plsc.VectorSubcoreMesh(core_axis_name="c", subcore_axis_name="s")

    def body(us_hbm, vs_hbm, src_hbm, dst_hbm, gu_hbm, gv_hbm, *scr):
        idxu = scr[0:NBUF]
        idxv = scr[NBUF:2 * NBUF]
        rowsu = scr[2 * NBUF:3 * NBUF]
        rowsv = scr[3 * NBUF:4 * NBUF]
        gsem = scr[4 * NBUF:5 * NBUF]
        wsem = scr[5 * NBUF:6 * NBUF]
        cid = lax.axis_index("c")
        sid = lax.axis_index("s")
        base = cid * (SC_TILES * EDGES_PER_TILE) + sid * EDGES_PER_TILE

        def g_start(b, off):
            pltpu.sync_copy(src_hbm.at[pl.ds(off, CHUNK)], idxu[b])
            pltpu.sync_copy(dst_hbm.at[pl.ds(off, CHUNK)], idxv[b])
            pltpu.async_copy(us_hbm.at[idxu[b]], rowsu[b], gsem[b])
            pltpu.async_copy(vs_hbm.at[idxv[b]], rowsv[b], gsem[b])

        def g_wait(b):
            pltpu.make_async_copy(us_hbm.at[idxu[b]], rowsu[b], gsem[b]).wait()
            pltpu.make_async_copy(vs_hbm.at[idxv[b]], rowsv[b], gsem[b]).wait()

        def w_start(b, off):
            pltpu.async_copy(rowsu[b], gu_hbm.at[pl.ds(off, CHUNK)], wsem[b])
            pltpu.async_copy(rowsv[b], gv_hbm.at[pl.ds(off, CHUNK)], wsem[b])

        def w_wait(b):
            pltpu.make_async_copy(rowsu[b], gu_hbm.at[pl.ds(0, CHUNK)],
                                  wsem[b]).wait()
            pltpu.make_async_copy(rowsv[b], gv_hbm.at[pl.ds(0, CHUNK)],
                                  wsem[b]).wait()

        for b in range(NBUF):
            g_start(b, base + b * CHUNK)

        def quad(k, carry):
            for b in range(NBUF):
                g_wait(b)
                w_start(b, base + ((k - 1) * NBUF + b) * CHUNK)
            for b in range(NBUF):
                w_wait(b)
                g_start(b, base + (k * NBUF + b) * CHUNK)
            return carry

        lax.fori_loop(1, NQUAD, quad, 0)
        for b in range(NBUF):
            g_wait(b)
            w_start(b, base + ((NQUAD - 1) * NBUF + b) * CHUNK)
        offt = base + NFULL * CHUNK
        w_wait(0)
        pltpu.sync_copy(src_hbm.at[pl.ds(offt, TAIL)], idxu[0].at[pl.ds(0, TAIL)])
        pltpu.sync_copy(dst_hbm.at[pl.ds(offt, TAIL)], idxv[0].at[pl.ds(0, TAIL)])
        cu = pltpu.async_copy(us_hbm.at[idxu[0].at[pl.ds(0, TAIL)]],
                              rowsu[0].at[pl.ds(0, TAIL)], gsem[0])
        cv = pltpu.async_copy(vs_hbm.at[idxv[0].at[pl.ds(0, TAIL)]],
                              rowsv[0].at[pl.ds(0, TAIL)], gsem[0])
        cu.wait()
        cv.wait()
        pltpu.sync_copy(rowsu[0].at[pl.ds(0, TAIL)], gu_hbm.at[pl.ds(offt, TAIL)])
        pltpu.sync_copy(rowsv[0].at[pl.ds(0, TAIL)], gv_hbm.at[pl.ds(offt, TAIL)])
        for b in range(1, NBUF):
            w_wait(b)

    return pl.kernel(
        body,
        out_type=(jax.ShapeDtypeStruct((NE, DL), F32),
                  jax.ShapeDtypeStruct((NE, DL), F32)),
        mesh=mesh,
        scratch_types=(
            [pltpu.VMEM((CHUNK,), jnp.int32)] * (2 * NBUF)
            + [pltpu.VMEM((CHUNK, DL), F32)] * (2 * NBUF)
            + [pltpu.SemaphoreType.DMA] * (2 * NBUF)
        ),
    )(us, vs, src, dst)


def _sc_scatter_add(ec, dst, zeros):
    mesh = plsc.VectorSubcoreMesh(core_axis_name="c", subcore_axis_name="s")

    def body(ec_hbm, dst_hbm, z_hbm, out_hbm, shared, *scr):
        idxb = scr[0:NBUF]
        rows = scr[NBUF:2 * NBUF]
        rsem = scr[2 * NBUF:3 * NBUF]
        ssem = scr[3 * NBUF:4 * NBUF]
        idxt, rowst = scr[4 * NBUF], scr[4 * NBUF + 1]
        cid = lax.axis_index("c")
        sid = lax.axis_index("s")

        @pl.when(sid < SC_TILES - 1)
        def _():
            pltpu.sync_copy(z_hbm.at[pl.ds(sid * NPT, NPT)],
                            shared.at[pl.ds(sid * NPT, NPT)])

        @pl.when(sid == SC_TILES - 1)
        def _():
            pltpu.sync_copy(z_hbm.at[pl.ds((SC_TILES - 1) * NPT, NPT_LAST)],
                            shared.at[pl.ds((SC_TILES - 1) * NPT, NPT_LAST)])

        plsc.subcore_barrier()
        base = cid * (SC_TILES * EDGES_PER_TILE) + sid * EDGES_PER_TILE

        def r_start(b, off):
            pltpu.sync_copy(dst_hbm.at[pl.ds(off, CHUNK)], idxb[b])
            pltpu.async_copy(ec_hbm.at[pl.ds(off, CHUNK)], rows[b], rsem[b])

        def r_wait(b):
            pltpu.make_async_copy(ec_hbm.at[pl.ds(0, CHUNK)], rows[b],
                                  rsem[b]).wait()

        def s_start(b):
            pltpu.async_copy(rows[b], shared.at[idxb[b]], ssem[b], add=True)

        def s_wait(b):
            pltpu.make_async_copy(rows[b], shared.at[idxb[b]], ssem[b]).wait()

        for b in range(NBUF):
            r_start(b, base + b * CHUNK)

        def quad(k, carry):
            for b in range(NBUF):
                r_wait(b)
                s_start(b)
            for b in range(NBUF):
                s_wait(b)
                r_start(b, base + (k * NBUF + b) * CHUNK)
            return carry

        lax.fori_loop(1, NQUAD, quad, 0)
        for b in range(NBUF):
            r_wait(b)
            s_start(b)
        for b in range(NBUF):
            s_wait(b)
        off = base + NFULL * CHUNK
        pltpu.sync_copy(dst_hbm.at[pl.ds(off, TAIL)], idxt)
        pltpu.sync_copy(ec_hbm.at[pl.ds(off, TAIL)], rowst)
        pltpu.sync_copy(rowst, shared.at[idxt], add=True)
        plsc.subcore_barrier()

        @pl.when(sid < SC_TILES - 1)
        def _():
            pltpu.sync_copy(shared.at[pl.ds(sid * NPT, NPT)],
                            out_hbm.at[pl.ds(cid * NN + sid * NPT, NPT)])

        @pl.when(sid == SC_TILES - 1)
        def _():
            pltpu.sync_copy(
                shared.at[pl.ds((SC_TILES - 1) * NPT, NPT_LAST)],
                out_hbm.at[pl.ds(cid * NN + (SC_TILES - 1) * NPT, NPT_LAST)])

    return pl.kernel(
        body,
        out_type=jax.ShapeDtypeStruct((2 * NN, DL), F32),
        mesh=mesh,
        scratch_types=(
            [pltpu.VMEM_SHARED((NN, DL), F32)]
            + [pltpu.VMEM((CHUNK,), jnp.int32)] * NBUF
            + [pltpu.VMEM((CHUNK, DL), F32)] * NBUF
            + [pltpu.SemaphoreType.DMA] * (2 * NBUF)
            + [pltpu.VMEM((TAIL,), jnp.int32),
               pltpu.VMEM((TAIL, DL), F32)]
        ),
    )(ec, dst, zeros)


def _row(a):
    return a.reshape(1, -1)


def _wspec():
    return pl.BlockSpec((DL, DL), lambda i: (0, 0))


def _rspec():
    return pl.BlockSpec((1, DL), lambda i: (0, 0))


def _sspec():
    return pl.BlockSpec((1, 1), lambda i: (0, 0))


def kernel(x, e, g, params, edges, node_idx, edge_idx, steps):
    del g, node_idx, edge_idx, steps
    p = params
    src = edges[0]
    dst = edges[1]

    Wce = p['core_edge']['W']
    A1, A2 = Wce[0:128], Wce[128:256]
    B1, B2 = Wce[256:384], Wce[384:512]
    C1, C2 = Wce[512:640], Wce[640:768]
    D2 = Wce[768:770]
    Wcn = p['core_node']['W']
    N1, N2, N3 = Wcn[0:128], Wcn[128:256], Wcn[256:384]
    ND = Wcn[384:386]

    beta_e = p['enc_glob']['bt'].reshape(1, 1)
    beta_c = p['core_glob']['bt'].reshape(1, 1)

    GE = NE // RE
    GN = NN // RN
    en = p['enc_node']
    us, vs, u0, v0, xp1, x0 = pl.pallas_call(
        _enc_node_body,
        grid=(GN,),
        in_specs=[pl.BlockSpec((RN, DL), lambda i: (i, 0)),
                  _wspec(), _rspec(), _rspec(), _rspec(),
                  _wspec(), _wspec(), _wspec(), _wspec(), _wspec(), _wspec()],
        out_specs=[pl.BlockSpec((RN, DL), lambda i: (i, 0))] * 6,
        out_shape=[jax.ShapeDtypeStruct((NN, DL), F32)] * 6,
    )(x, en['W'], _row(en['b']), _row(en['g']), _row(en['bt']),
      B1, B2, C1, C2, N1, N2)

    ee = p['enc_edge']
    zeros = jnp.zeros((NN, DL), F32)
    ce = p['core_edge']
    de0, de1, oe_p = p['dec_edge0'], p['dec_edge1'], p['out_edge']
    cn = p['core_node']
    dn0, dn1, on_p = p['dec_node0'], p['dec_node1'], p['out_node']

    espec = pl.BlockSpec((RE, DL), lambda i: (i, 0))

    def edge_step(first, gu, gv, scal2, base=None):
        if first:
            in_specs = [pl.BlockSpec((16, RE), lambda i: (0, i)), espec, espec,
                        pl.BlockSpec((16, DL), lambda i: (0, 0)),
                        _rspec(), _rspec(), _rspec(), _wspec(), _wspec()]
            args = [e.T, gu, gv, ee['W'], _row(ee['b']), _row(ee['g']),
                    _row(ee['bt']), A1, A2]
        else:
            in_specs = [espec, espec, espec]
            args = [base, gu, gv]
        in_specs += [pl.BlockSpec((2, DL), lambda i: (0, 0)),
                     _sspec(), _sspec(), _rspec(), _rspec(), _rspec(),
                     _wspec(), _rspec(), _rspec(), _rspec(),
                     _wspec(), _rspec(), _rspec(), _rspec(),
                     _rspec(), _sspec()]
        args += [D2, beta_e, scal2,
                 _row(ce['b']), _row(ce['g']), _row(ce['bt']),
                 de0['W'], _row(de0['b']), _row(de0['g']), _row(de0['bt']),
                 de1['W'], _row(de1['b']), _row(de1['g']), _row(de1['bt']),
                 oe_p['W'].reshape(1, DL), oe_p['b'].reshape(1, 1)]
        out_specs = [espec, pl.BlockSpec((1, RE), lambda i: (0, i))]
        out_shape = [jax.ShapeDtypeStruct((NE, DL), F32),
                     jax.ShapeDtypeStruct((1, NE), F32)]
        if first:
            out_specs += [espec]
            out_shape += [jax.ShapeDtypeStruct((NE, DL), F32)]
        return pl.pallas_call(
            functools.partial(_edge_step_body, first),
            grid=(GE,),
            in_specs=in_specs,
            out_specs=out_specs,
            out_shape=out_shape,
        )(*args)

    def node_step(first, base, aggp, scal2, xcur=None):
        nspec = pl.BlockSpec((RN, DL), lambda i: (i, 0))
        in_specs = [nspec]
        args = [base]
        if not first:
            in_specs += [nspec, _wspec()]
            args += [xcur, N2]
        in_specs += [nspec, nspec, _wspec(),
                     pl.BlockSpec((2, DL), lambda i: (0, 0)),
                     _sspec(), _sspec(),
                     _rspec(), _rspec(), _rspec(),
                     _wspec(), _rspec(), _rspec(), _rspec(),
                     _wspec(), _rspec(), _rspec(), _rspec(),
                     pl.BlockSpec((DL, 1), lambda i: (0, 0)), _sspec()]
        args += [aggp[:NN], aggp[NN:], N3, ND, beta_e, scal2,
                 _row(cn['b']), _row(cn['g']), _row(cn['bt']),
                 dn0['W'], _row(dn0['b']), _row(dn0['g']), _row(dn0['bt']),
                 dn1['W'], _row(dn1['b']), _row(dn1['g']), _row(dn1['bt']),
                 on_p['W'], on_p['b'].reshape(1, 1)]
        if first:
            in_specs += [nspec, nspec, _wspec(), _wspec()]
            args += [u0, v0, B2, C2]
            out_specs = [nspec, pl.BlockSpec((RN, 1), lambda i: (i, 0)),
                         nspec, nspec]
            out_shape = [jax.ShapeDtypeStruct((NN, DL), F32),
                         jax.ShapeDtypeStruct((NN, 1), F32),
                         jax.ShapeDtypeStruct((NN, DL), F32),
                         jax.ShapeDtypeStruct((NN, DL), F32)]
        else:
            out_specs = [pl.BlockSpec((RN, 1), lambda i: (i, 0))]
            out_shape = [jax.ShapeDtypeStruct((NN, 1), F32)]
        return pl.pallas_call(
            functools.partial(_node_step_body, first),
            grid=(GN,),
            in_specs=in_specs,
            out_specs=out_specs,
            out_shape=out_shape,
        )(*args)

    gu1, gv1 = _sc_gather(us, vs, src, dst)
    ecur2, oe1t, p2 = edge_step(True, gu1, gv1, beta_e)
    aggp1 = _sc_scatter_add(ecur2, dst, zeros)
    xcur2, ox1, us2, vs2 = node_step(True, xp1, aggp1, beta_e)

    gu2, gv2 = _sc_gather(us2, vs2, src, dst)
    ecur3, oe2t = edge_step(False, gu2, gv2, beta_c, base=p2)
    aggp2 = _sc_scatter_add(ecur3, dst, zeros)
    (ox2,) = node_step(False, x0, aggp2, beta_c, xcur=xcur2)
    oe1 = oe1t.reshape(NE, 1)
    oe2 = oe2t.reshape(NE, 1)

    og_p = p['out_glob']
    og = pl.pallas_call(
        _og_body,
        in_specs=[pl.BlockSpec(memory_space=pltpu.VMEM)] * 3,
        out_shape=jax.ShapeDtypeStruct((NG, 1), F32),
    )(p['dec_glob']['bt'].reshape(1, 1), og_p['W'], og_p['b'].reshape(1, 1))

    return (ox1, oe1, og, ox2, oe2, og)


# speedup vs baseline: 1.1274x; 1.1237x over previous
import functools

import jax
import jax.numpy as jnp
from jax import lax
from jax.experimental import pallas as pl
from jax.experimental.pallas import tpu as pltpu
from jax.experimental.pallas import tpu_sc as plsc

NE = 160000
NN = 10000
DL = 128
NG = 32
EPS = 1e-5
F32 = jnp.float32
PREC = None

SC_CORES = 2
SC_TILES = 16
EDGES_PER_TILE = NE // (SC_CORES * SC_TILES)
CHUNK = 96
HALF = NE // 2
HPT = 2496
HNFULL = 26
HNBUF = 2
HNPAIR = HNFULL // HNBUF
HEXTRA = 128
NPT = 624
NPT_LAST = NN - NPT * (SC_TILES - 1)

RE = 3200
RN = 2000


def _ln_rows(h, g, bt):
    m = jnp.mean(h, axis=-1, keepdims=True)
    d = h - m
    v = jnp.mean(d * d, axis=-1, keepdims=True)
    return d / jnp.sqrt(v + EPS) * g + bt


def _mlp(v, w_ref, b_ref, g_ref, bt_ref):
    h = jnp.dot((v), (w_ref[...]), preferred_element_type=F32, precision=PREC) + b_ref[...]
    return _ln_rows(jnp.maximum(h, 0.0), g_ref[...], bt_ref[...])


def _edge_step_body(first, *refs):
    if first:
        (et_ref, gu_ref, gv_ref, we_ref, be_ref, ge_ref, bte_ref,
         a1_ref, a2_ref, d2_ref, s1_ref, s2_ref, bce_ref, gce_ref, btce_ref,
         w0_ref, b0_ref, g0_ref, bt0_ref, w1_ref, b1_ref, g1_ref, bt1_ref,
         wo_ref, bo_ref, ec_ref, oe_ref, p2_ref) = refs
        h = lax.dot_general((et_ref[...]), (we_ref[...]), (((0,), (0,)), ((), ())),
                            preferred_element_type=F32, precision=PREC) + be_ref[...]
        el0 = _ln_rows(jnp.maximum(h, 0.0), ge_ref[...], bte_ref[...])
        t1 = jnp.dot((el0), (a1_ref[...]), preferred_element_type=F32, precision=PREC)
        pre = t1 + jnp.dot((el0), (a2_ref[...]), preferred_element_type=F32, precision=PREC)
    else:
        (base_ref, gu_ref, gv_ref, d2_ref, s1_ref, s2_ref,
         bce_ref, gce_ref, btce_ref,
         w0_ref, b0_ref, g0_ref, bt0_ref, w1_ref, b1_ref, g1_ref, bt1_ref,
         wo_ref, bo_ref, ec_ref, oe_ref) = refs
        pre = base_ref[...]
    bias = s1_ref[...] * d2_ref[0:1, :] + s2_ref[...] * d2_ref[1:2, :]
    pre = pre + gu_ref[...] + gv_ref[...] + bias + bce_ref[...]
    ec = _ln_rows(jnp.maximum(pre, 0.0), gce_ref[...], btce_ref[...])
    ec_ref[...] = ec
    d0 = _mlp(ec, w0_ref, b0_ref, g0_ref, bt0_ref)
    d1 = _mlp(d0, w1_ref, b1_ref, g1_ref, bt1_ref)
    oe_ref[...] = lax.dot_general((wo_ref[...]), (d1), (((1,), (1,)), ((), ())),
                                  preferred_element_type=F32,
                                  precision=PREC) + bo_ref[...]
    if first:
        p2_ref[...] = t1 + jnp.dot((ec), (a2_ref[...]), preferred_element_type=F32, precision=PREC)


def _enc_node_body(x_ref, w_ref, b_ref, g_ref, bt_ref,
                   b1_ref, b2_ref, c1_ref, c2_ref, n1_ref, n2_ref,
                   us1_ref, vs1_ref, u0_ref, v0_ref, xp1_ref, x0_ref):
    xl0 = _mlp(x_ref[...], w_ref, b_ref, g_ref, bt_ref)

    def dot(w_ref):
        return jnp.dot((xl0), (w_ref[...]), preferred_element_type=F32,
                       precision=PREC)

    u0 = dot(b1_ref)
    v0 = dot(c1_ref)
    x0 = dot(n1_ref)
    u0_ref[...] = u0
    v0_ref[...] = v0
    x0_ref[...] = x0
    us1_ref[...] = u0 + dot(b2_ref)
    vs1_ref[...] = v0 + dot(c2_ref)
    xp1_ref[...] = x0 + dot(n2_ref)


def _node_step_body(first, *refs):
    if first:
        (base_ref, a0_ref, a1_ref, a2_ref, a3_ref, n3_ref, nd_ref, s1_ref, s2_ref,
         bcn_ref, gcn_ref, btcn_ref, w0_ref, b0_ref, g0_ref, bt0_ref,
         w1_ref, b1_ref, g1_ref, bt1_ref, wo_ref, bo_ref,
         u0_ref, v0_ref, b2w_ref, c2w_ref,
         xout_ref, ox_ref, us_ref, vs_ref) = refs
        pre = base_ref[...]
    else:
        (base_ref, xc_ref, n2_ref, a0_ref, a1_ref, a2_ref, a3_ref, n3_ref, nd_ref,
         s1_ref, s2_ref, bcn_ref, gcn_ref, btcn_ref,
         w0_ref, b0_ref, g0_ref, bt0_ref, w1_ref, b1_ref, g1_ref, bt1_ref,
         wo_ref, bo_ref, ox_ref) = refs
        pre = base_ref[...] + jnp.dot((xc_ref[...]), (n2_ref[...]),
                                      preferred_element_type=F32, precision=PREC)
    agg = (a0_ref[...] + a1_ref[...]) + (a2_ref[...] + a3_ref[...])
    bias = s1_ref[...] * nd_ref[0:1, :] + s2_ref[...] * nd_ref[1:2, :]
    pre = pre + jnp.dot((agg), (n3_ref[...]), preferred_element_type=F32, precision=PREC)
    pre = pre + bias + bcn_ref[...]
    xc = _ln_rows(jnp.maximum(pre, 0.0), gcn_ref[...], btcn_ref[...])
    d0 = _mlp(xc, w0_ref, b0_ref, g0_ref, bt0_ref)
    d1 = _mlp(d0, w1_ref, b1_ref, g1_ref, bt1_ref)
    ox_ref[...] = jnp.dot((d1), (wo_ref[...]), preferred_element_type=F32, precision=PREC) + bo_ref[...]
    if first:
        xout_ref[...] = xc
        us_ref[...] = u0_ref[...] + jnp.dot((xc), (b2w_ref[...]),
                                            preferred_element_type=F32, precision=PREC)
        vs_ref[...] = v0_ref[...] + jnp.dot((xc), (c2w_ref[...]),
                                            preferred_element_type=F32, precision=PREC)


def _og_body(btd_ref, wg_ref, bg_ref, og_ref):
    val = btd_ref[...] * wg_ref[...] + bg_ref[...]
    og_ref[...] = jnp.broadcast_to(val, (NG, 1))


def _sc_gather(us, vs, src, dst, half):
    mesh = plsc.VectorSubcoreMesh(core_axis_name="c", subcore_axis_name="s")

    def body(us_hbm, vs_hbm, src_hbm, dst_hbm, gu_hbm, gv_hbm, *scr):
        idxu = scr[0:HNBUF]
        idxv = scr[HNBUF:2 * HNBUF]
        rowsu = scr[2 * HNBUF:3 * HNBUF]
        rowsv = scr[3 * HNBUF:4 * HNBUF]
        gsem = scr[4 * HNBUF:5 * HNBUF]
        wsem = scr[5 * HNBUF:6 * HNBUF]
        idxux, idxvx, rowsux, rowsvx = scr[6 * HNBUF:6 * HNBUF + 4]
        cid = lax.axis_index("c")
        sid = lax.axis_index("s")
        wid = cid * SC_TILES + sid
        base = wid * HPT

        def g_start(b, off):
            pltpu.sync_copy(src_hbm.at[pl.ds(half * HALF + off, CHUNK)], idxu[b])
            pltpu.sync_copy(dst_hbm.at[pl.ds(half * HALF + off, CHUNK)], idxv[b])
            pltpu.async_copy(us_hbm.at[idxu[b]], rowsu[b], gsem[b])
            pltpu.async_copy(vs_hbm.at[idxv[b]], rowsv[b], gsem[b])

        def g_wait(b):
            pltpu.make_async_copy(us_hbm.at[idxu[b]], rowsu[b], gsem[b]).wait()
            pltpu.make_async_copy(vs_hbm.at[idxv[b]], rowsv[b], gsem[b]).wait()

        def w_start(b, off):
            pltpu.async_copy(rowsu[b], gu_hbm.at[pl.ds(off, CHUNK)], wsem[b])
            pltpu.async_copy(rowsv[b], gv_hbm.at[pl.ds(off, CHUNK)], wsem[b])

        def w_wait(b):
            pltpu.make_async_copy(rowsu[b], gu_hbm.at[pl.ds(0, CHUNK)],
                                  wsem[b]).wait()
            pltpu.make_async_copy(rowsv[b], gv_hbm.at[pl.ds(0, CHUNK)],
                                  wsem[b]).wait()

        for b in range(HNBUF):
            g_start(b, base + b * CHUNK)

        def pair(k, carry):
            for b in range(HNBUF):
                g_wait(b)
                w_start(b, base + ((k - 1) * HNBUF + b) * CHUNK)
            for b in range(HNBUF):
                w_wait(b)
                g_start(b, base + (k * HNBUF + b) * CHUNK)
            return carry

        lax.fori_loop(1, HNPAIR, pair, 0)
        for b in range(HNBUF):
            g_wait(b)
            w_start(b, base + ((HNPAIR - 1) * HNBUF + b) * CHUNK)

        @pl.when(wid == SC_CORES * SC_TILES - 1)
        def _():
            loff = (SC_CORES * SC_TILES - 1) * HPT + HNFULL * CHUNK
            pltpu.sync_copy(src_hbm.at[pl.ds(half * HALF + loff, HEXTRA)], idxux)
            pltpu.sync_copy(dst_hbm.at[pl.ds(half * HALF + loff, HEXTRA)], idxvx)
            cu = pltpu.async_copy(us_hbm.at[idxux], rowsux, gsem[0])
            cv = pltpu.async_copy(vs_hbm.at[idxvx], rowsvx, gsem[0])
            cu.wait()
            cv.wait()
            pltpu.sync_copy(rowsux, gu_hbm.at[pl.ds(loff, HEXTRA)])
            pltpu.sync_copy(rowsvx, gv_hbm.at[pl.ds(loff, HEXTRA)])

        for b in range(HNBUF):
            w_wait(b)

    return pl.kernel(
        body,
        out_type=(jax.ShapeDtypeStruct((HALF, DL), F32),
                  jax.ShapeDtypeStruct((HALF, DL), F32)),
        mesh=mesh,
        scratch_types=(
            [pltpu.VMEM((CHUNK,), jnp.int32)] * (2 * HNBUF)
            + [pltpu.VMEM((CHUNK, DL), F32)] * (2 * HNBUF)
            + [pltpu.SemaphoreType.DMA] * (2 * HNBUF)
            + [pltpu.VMEM((HEXTRA,), jnp.int32)] * 2
            + [pltpu.VMEM((HEXTRA, DL), F32)] * 2
        ),
    )(us, vs, src, dst)


def _sc_scatter_add(ec, dst, zeros, half):
    mesh = plsc.VectorSubcoreMesh(core_axis_name="c", subcore_axis_name="s")

    def body(ec_hbm, dst_hbm, z_hbm, out_hbm, shared, *scr):
        idxb = scr[0:HNBUF]
        rows = scr[HNBUF:2 * HNBUF]
        rsem = scr[2 * HNBUF:3 * HNBUF]
        ssem = scr[3 * HNBUF:4 * HNBUF]
        idxt, rowst = scr[4 * HNBUF], scr[4 * HNBUF + 1]
        cid = lax.axis_index("c")
        sid = lax.axis_index("s")
        wid = cid * SC_TILES + sid

        @pl.when(sid < SC_TILES - 1)
        def _():
            pltpu.sync_copy(z_hbm.at[pl.ds(sid * NPT, NPT)],
                            shared.at[pl.ds(sid * NPT, NPT)])

        @pl.when(sid == SC_TILES - 1)
        def _():
            pltpu.sync_copy(z_hbm.at[pl.ds((SC_TILES - 1) * NPT, NPT_LAST)],
                            shared.at[pl.ds((SC_TILES - 1) * NPT, NPT_LAST)])

        plsc.subcore_barrier()
        base = wid * HPT

        def r_start(b, off):
            pltpu.sync_copy(dst_hbm.at[pl.ds(half * HALF + off, CHUNK)], idxb[b])
            pltpu.async_copy(ec_hbm.at[pl.ds(off, CHUNK)], rows[b], rsem[b])

        def r_wait(b):
            pltpu.make_async_copy(ec_hbm.at[pl.ds(0, CHUNK)], rows[b],
                                  rsem[b]).wait()

        def s_start(b):
            pltpu.async_copy(rows[b], shared.at[idxb[b]], ssem[b], add=True)

        def s_wait(b):
            pltpu.make_async_copy(rows[b], shared.at[idxb[b]], ssem[b]).wait()

        for b in range(HNBUF):
            r_start(b, base + b * CHUNK)

        def pair(k, carry):
            for b in range(HNBUF):
                r_wait(b)
                s_start(b)
            for b in range(HNBUF):
                s_wait(b)
                r_start(b, base + (k * HNBUF + b) * CHUNK)
            return carry

        lax.fori_loop(1, HNPAIR, pair, 0)
        for b in range(HNBUF):
            r_wait(b)
            s_start(b)
        for b in range(HNBUF):
            s_wait(b)
        @pl.when(wid == SC_CORES * SC_TILES - 1)
        def _():
            loff = (SC_CORES * SC_TILES - 1) * HPT + HNFULL * CHUNK
            pltpu.sync_copy(dst_hbm.at[pl.ds(half * HALF + loff, HEXTRA)], idxt)
            pltpu.sync_copy(ec_hbm.at[pl.ds(loff, HEXTRA)], rowst)
            pltpu.sync_copy(rowst, shared.at[idxt], add=True)

        plsc.subcore_barrier()

        @pl.when(sid < SC_TILES - 1)
        def _():
            pltpu.sync_copy(shared.at[pl.ds(sid * NPT, NPT)],
                            out_hbm.at[pl.ds(cid * NN + sid * NPT, NPT)])

        @pl.when(sid == SC_TILES - 1)
        def _():
            pltpu.sync_copy(
                shared.at[pl.ds((SC_TILES - 1) * NPT, NPT_LAST)],
                out_hbm.at[pl.ds(cid * NN + (SC_TILES - 1) * NPT, NPT_LAST)])

    return pl.kernel(
        body,
        out_type=jax.ShapeDtypeStruct((2 * NN, DL), F32),
        mesh=mesh,
        scratch_types=(
            [pltpu.VMEM_SHARED((NN, DL), F32)]
            + [pltpu.VMEM((CHUNK,), jnp.int32)] * HNBUF
            + [pltpu.VMEM((CHUNK, DL), F32)] * HNBUF
            + [pltpu.SemaphoreType.DMA] * (2 * HNBUF)
            + [pltpu.VMEM((HEXTRA,), jnp.int32),
               pltpu.VMEM((HEXTRA, DL), F32)]
        ),
    )(ec, dst, zeros)


def _row(a):
    return a.reshape(1, -1)


def _wspec():
    return pl.BlockSpec((DL, DL), lambda i: (0, 0))


def _rspec():
    return pl.BlockSpec((1, DL), lambda i: (0, 0))


def _sspec():
    return pl.BlockSpec((1, 1), lambda i: (0, 0))


def kernel(x, e, g, params, edges, node_idx, edge_idx, steps):
    del g, node_idx, edge_idx, steps
    p = params
    src = edges[0]
    dst = edges[1]

    Wce = p['core_edge']['W']
    A1, A2 = Wce[0:128], Wce[128:256]
    B1, B2 = Wce[256:384], Wce[384:512]
    C1, C2 = Wce[512:640], Wce[640:768]
    D2 = Wce[768:770]
    Wcn = p['core_node']['W']
    N1, N2, N3 = Wcn[0:128], Wcn[128:256], Wcn[256:384]
    ND = Wcn[384:386]

    beta_e = p['enc_glob']['bt'].reshape(1, 1)
    beta_c = p['core_glob']['bt'].reshape(1, 1)

    GE = NE // RE
    GN = NN // RN
    en = p['enc_node']
    us, vs, u0, v0, xp1, x0 = pl.pallas_call(
        _enc_node_body,
        grid=(GN,),
        in_specs=[pl.BlockSpec((RN, DL), lambda i: (i, 0)),
                  _wspec(), _rspec(), _rspec(), _rspec(),
                  _wspec(), _wspec(), _wspec(), _wspec(), _wspec(), _wspec()],
        out_specs=[pl.BlockSpec((RN, DL), lambda i: (i, 0))] * 6,
        out_shape=[jax.ShapeDtypeStruct((NN, DL), F32)] * 6,
    )(x, en['W'], _row(en['b']), _row(en['g']), _row(en['bt']),
      B1, B2, C1, C2, N1, N2)

    ee = p['enc_edge']
    zeros = jnp.zeros((NN, DL), F32)
    ce = p['core_edge']
    de0, de1, oe_p = p['dec_edge0'], p['dec_edge1'], p['out_edge']
    cn = p['core_node']
    dn0, dn1, on_p = p['dec_node0'], p['dec_node1'], p['out_node']

    espec = pl.BlockSpec((RE, DL), lambda i: (i, 0))

    def edge_step(first, gu, gv, scal2, half, base=None):
        hoff = half * (HALF // RE)
        if first:
            in_specs = [pl.BlockSpec((16, RE), lambda i, h=hoff: (0, i + h)),
                        espec, espec,
                        pl.BlockSpec((16, DL), lambda i: (0, 0)),
                        _rspec(), _rspec(), _rspec(), _wspec(), _wspec()]
            args = [e.T, gu, gv, ee['W'], _row(ee['b']), _row(ee['g']),
                    _row(ee['bt']), A1, A2]
        else:
            in_specs = [espec, espec, espec]
            args = [base, gu, gv]
        in_specs += [pl.BlockSpec((2, DL), lambda i: (0, 0)),
                     _sspec(), _sspec(), _rspec(), _rspec(), _rspec(),
                     _wspec(), _rspec(), _rspec(), _rspec(),
                     _wspec(), _rspec(), _rspec(), _rspec(),
                     _rspec(), _sspec()]
        args += [D2, beta_e, scal2,
                 _row(ce['b']), _row(ce['g']), _row(ce['bt']),
                 de0['W'], _row(de0['b']), _row(de0['g']), _row(de0['bt']),
                 de1['W'], _row(de1['b']), _row(de1['g']), _row(de1['bt']),
                 oe_p['W'].reshape(1, DL), oe_p['b'].reshape(1, 1)]
        out_specs = [espec, pl.BlockSpec((1, RE), lambda i: (0, i))]
        out_shape = [jax.ShapeDtypeStruct((HALF, DL), F32),
                     jax.ShapeDtypeStruct((1, HALF), F32)]
        if first:
            out_specs += [espec]
            out_shape += [jax.ShapeDtypeStruct((HALF, DL), F32)]
        return pl.pallas_call(
            functools.partial(_edge_step_body, first),
            grid=(HALF // RE,),
            in_specs=in_specs,
            out_specs=out_specs,
            out_shape=out_shape,
        )(*args)

    def node_step(first, base, aggp_a, aggp_b, scal2, xcur=None):
        nspec = pl.BlockSpec((RN, DL), lambda i: (i, 0))
        in_specs = [nspec]
        args = [base]
        if not first:
            in_specs += [nspec, _wspec()]
            args += [xcur, N2]
        in_specs += [nspec, nspec, nspec, nspec, _wspec(),
                     pl.BlockSpec((2, DL), lambda i: (0, 0)),
                     _sspec(), _sspec(),
                     _rspec(), _rspec(), _rspec(),
                     _wspec(), _rspec(), _rspec(), _rspec(),
                     _wspec(), _rspec(), _rspec(), _rspec(),
                     pl.BlockSpec((DL, 1), lambda i: (0, 0)), _sspec()]
        args += [aggp_a[:NN], aggp_a[NN:], aggp_b[:NN], aggp_b[NN:],
                 N3, ND, beta_e, scal2,
                 _row(cn['b']), _row(cn['g']), _row(cn['bt']),
                 dn0['W'], _row(dn0['b']), _row(dn0['g']), _row(dn0['bt']),
                 dn1['W'], _row(dn1['b']), _row(dn1['g']), _row(dn1['bt']),
                 on_p['W'], on_p['b'].reshape(1, 1)]
        if first:
            in_specs += [nspec, nspec, _wspec(), _wspec()]
            args += [u0, v0, B2, C2]
            out_specs = [nspec, pl.BlockSpec((RN, 1), lambda i: (i, 0)),
                         nspec, nspec]
            out_shape = [jax.ShapeDtypeStruct((NN, DL), F32),
                         jax.ShapeDtypeStruct((NN, 1), F32),
                         jax.ShapeDtypeStruct((NN, DL), F32),
                         jax.ShapeDtypeStruct((NN, DL), F32)]
        else:
            out_specs = [pl.BlockSpec((RN, 1), lambda i: (i, 0))]
            out_shape = [jax.ShapeDtypeStruct((NN, 1), F32)]
        return pl.pallas_call(
            functools.partial(_node_step_body, first),
            grid=(GN,),
            in_specs=in_specs,
            out_specs=out_specs,
            out_shape=out_shape,
        )(*args)

    gu1a, gv1a = _sc_gather(us, vs, src, dst, 0)
    gu1b, gv1b = _sc_gather(us, vs, src, dst, 1)
    ec2a, oe1at, p2a = edge_step(True, gu1a, gv1a, beta_e, 0)
    agg1a = _sc_scatter_add(ec2a, dst, zeros, 0)
    ec2b, oe1bt, p2b = edge_step(True, gu1b, gv1b, beta_e, 1)
    agg1b = _sc_scatter_add(ec2b, dst, zeros, 1)
    xcur2, ox1, us2, vs2 = node_step(True, xp1, agg1a, agg1b, beta_e)

    gu2a, gv2a = _sc_gather(us2, vs2, src, dst, 0)
    gu2b, gv2b = _sc_gather(us2, vs2, src, dst, 1)
    ec3a, oe2at = edge_step(False, gu2a, gv2a, beta_c, 0, base=p2a)
    agg2a = _sc_scatter_add(ec3a, dst, zeros, 0)
    ec3b, oe2bt = edge_step(False, gu2b, gv2b, beta_c, 1, base=p2b)
    agg2b = _sc_scatter_add(ec3b, dst, zeros, 1)
    (ox2,) = node_step(False, x0, agg2a, agg2b, beta_c, xcur=xcur2)
    oe1 = jnp.concatenate([oe1at, oe1bt], axis=1).reshape(NE, 1)
    oe2 = jnp.concatenate([oe2at, oe2bt], axis=1).reshape(NE, 1)

    og_p = p['out_glob']
    og = pl.pallas_call(
        _og_body,
        in_specs=[pl.BlockSpec(memory_space=pltpu.VMEM)] * 3,
        out_shape=jax.ShapeDtypeStruct((NG, 1), F32),
    )(p['dec_glob']['bt'].reshape(1, 1), og_p['W'], og_p['b'].reshape(1, 1))

    return (ox1, oe1, og, ox2, oe2, og)
